# baseline (speedup 1.0000x reference)
import functools

import jax
import jax.numpy as jnp
import numpy as np
from jax import lax
from jax.experimental import pallas as pl
from jax.experimental.pallas import tpu as pltpu
from jax.experimental.pallas import tpu_sc as plsc

NC, NS, LANES = 2, 16, 16
NW = NC * NS
ROWS = 100000
SIZE_IN_K = 512
SIZE_OUT_K = 128

ROWS_SC = 37000
SLABS = ROWS_SC // 8
SLABS_BASE = SLABS // NW
EXTRA = SLABS - SLABS_BASE * NW
assert ROWS_SC % 8 == 0 and SLABS_BASE % 3 == 0 and 0 <= EXTRA < NW
ROWS_TC = ROWS - ROWS_SC
BLOCK_R = 1000
assert ROWS_TC % BLOCK_R == 0

R_CHUNK = 24
NCHUNK = SLABS_BASE // 3
VPER = R_CHUNK * SIZE_OUT_K // LANES
VPER_TAIL = 8 * SIZE_OUT_K // LANES

_MESH = plsc.VectorSubcoreMesh(core_axis_name="c", subcore_axis_name="s")


@functools.partial(
    pl.kernel,
    out_type=jax.ShapeDtypeStruct((ROWS_SC, SIZE_OUT_K), jnp.float32),
    mesh=_MESH,
    compiler_params=pltpu.CompilerParams(needs_layout_passes=False),
    scratch_types=[
        pltpu.VMEM((R_CHUNK, SIZE_IN_K), jnp.float32),
        pltpu.VMEM((R_CHUNK, SIZE_IN_K), jnp.float32),
        pltpu.VMEM((R_CHUNK, SIZE_OUT_K), jnp.float32),
        pltpu.VMEM((R_CHUNK, SIZE_OUT_K), jnp.float32),
        pltpu.SemaphoreType.DMA,
        pltpu.SemaphoreType.DMA,
        pltpu.SemaphoreType.DMA,
        pltpu.SemaphoreType.DMA,
    ],
)
def _sc_seg_sum(x_hbm, out_hbm, in0, in1, o0, o1, si0, si1, so0, so1):
    ins, outs = [in0, in1], [o0, o1]
    sis, sos = [si0, si1], [so0, so1]
    wid = lax.axis_index("s") * NC + lax.axis_index("c")
    s0 = SLABS_BASE * wid + jnp.minimum(wid, EXTRA)
    r0 = s0 * 8
    has_tail = wid < EXTRA
    lane = lax.iota(jnp.int32, 16)
    quad = lax.shift_right_logical(lane, 2)
    cphase = [lane * 4 + ((quad + g) & 3) for g in range(4)]

    def in_copy(i, b):
        return pltpu.make_async_copy(
            x_hbm.at[pl.ds(r0 + i * R_CHUNK, R_CHUNK)], ins[b], sis[b])

    def out_copy(i, b):
        return pltpu.make_async_copy(
            outs[b], out_hbm.at[pl.ds(r0 + i * R_CHUNK, R_CHUNK)], sos[b])

    def compute(b, nv):
        in_v, out_v = ins[b], outs[b]

        @plsc.parallel_loop(0, nv, 1, unroll=8)
        def step(v):
            row = v >> 3
            j = v & 7
            ridx = jnp.full((16,), row, jnp.int32)
            coff = j * 64
            a = (plsc.load_gather(in_v, [ridx, cphase[0] + coff])
                 + plsc.load_gather(in_v, [ridx, cphase[1] + coff])
                 + plsc.load_gather(in_v, [ridx, cphase[2] + coff])
                 + plsc.load_gather(in_v, [ridx, cphase[3] + coff]))
            out_v[row, pl.ds(j * 16, 16)] = a

    in_copy(0, 0).start()
    in_copy(1, 1).start()

    def pair(g2, carry):
        for b in range(2):
            i = 2 * g2 + b

            @pl.when(i >= 2)
            def _wait_outbuf():
                out_copy(i - 2, b).wait()

            in_copy(i, b).wait()
            compute(b, VPER)
            out_copy(i, b).start()

            @pl.when(i + 2 < NCHUNK)
            def _prefetch():
                in_copy(i + 2, b).start()

        return carry

    lax.fori_loop(0, NCHUNK // 2, pair, 0)

    out_copy(NCHUNK - 2, 0).wait()

    @pl.when(has_tail)
    def _tail():
        tr0 = r0 + NCHUNK * R_CHUNK
        tin = pltpu.make_async_copy(
            x_hbm.at[pl.ds(tr0, 8)], ins[0].at[pl.ds(0, 8)], sis[0])
        tin.start()
        tin.wait()
        compute(0, VPER_TAIL)
        tout = pltpu.make_async_copy(
            outs[0].at[pl.ds(0, 8)], out_hbm.at[pl.ds(tr0, 8)], sos[0])
        tout.start()
        tout.wait()

    out_copy(NCHUNK - 1, 1).wait()


_SEL = np.zeros((SIZE_IN_K, SIZE_OUT_K), np.float32)
_SEL[np.arange(SIZE_IN_K), np.arange(SIZE_IN_K) // 4] = 1.0


def _tc_body(x_ref, s_ref, o_ref):
    o_ref[...] = jnp.dot(x_ref[...], s_ref[...],
                         preferred_element_type=jnp.float32)


_tc_seg_sum = pl.pallas_call(
    _tc_body,
    grid=(ROWS_TC // BLOCK_R,),
    in_specs=[
        pl.BlockSpec((BLOCK_R, SIZE_IN_K), lambda i: (i, 0)),
        pl.BlockSpec((SIZE_IN_K, SIZE_OUT_K), lambda i: (0, 0)),
    ],
    out_specs=pl.BlockSpec((BLOCK_R, SIZE_OUT_K), lambda i: (i, 0)),
    out_shape=jax.ShapeDtypeStruct((ROWS_TC, SIZE_OUT_K), jnp.float32),
)


def kernel(x):
    out_sc = _sc_seg_sum(x[ROWS_TC:])
    out_tc = _tc_seg_sum(x[:ROWS_TC], jnp.asarray(_SEL))
    return jnp.concatenate([out_tc, out_sc], axis=0)


# speedup vs baseline: 2.0197x; 2.0197x over previous
import functools

import jax
import jax.numpy as jnp
import numpy as np
from jax import lax
from jax.experimental import pallas as pl
from jax.experimental.pallas import tpu as pltpu
from jax.experimental.pallas import tpu_sc as plsc

NC, NS, LANES = 2, 16, 16
NW = NC * NS
ROWS = 100000
SIZE_IN_K = 512
SIZE_OUT_K = 128

ROWS_SC = 37000
SLABS = ROWS_SC // 8
SLABS_BASE = SLABS // NW
EXTRA = SLABS - SLABS_BASE * NW
assert ROWS_SC % 8 == 0 and SLABS_BASE % 3 == 0 and 0 <= EXTRA < NW
ROWS_TC = ROWS - ROWS_SC
BLOCK_R = 1000
assert ROWS_TC % BLOCK_R == 0

R_CHUNK = 24
NCHUNK = SLABS_BASE // 3
VPER = R_CHUNK * SIZE_OUT_K // LANES
VPER_TAIL = 8 * SIZE_OUT_K // LANES

_MESH = plsc.VectorSubcoreMesh(core_axis_name="c", subcore_axis_name="s")


@functools.partial(
    pl.kernel,
    out_type=jax.ShapeDtypeStruct((ROWS_SC, SIZE_OUT_K), jnp.float32),
    mesh=_MESH,
    compiler_params=pltpu.CompilerParams(needs_layout_passes=False),
    scratch_types=[
        pltpu.VMEM((R_CHUNK, SIZE_IN_K), jnp.float32),
        pltpu.VMEM((R_CHUNK, SIZE_IN_K), jnp.float32),
        pltpu.VMEM((R_CHUNK, SIZE_OUT_K), jnp.float32),
        pltpu.VMEM((R_CHUNK, SIZE_OUT_K), jnp.float32),
        pltpu.SemaphoreType.DMA,
        pltpu.SemaphoreType.DMA,
        pltpu.SemaphoreType.DMA,
        pltpu.SemaphoreType.DMA,
    ],
)
def _sc_seg_sum(x_hbm, out_hbm, in0, in1, o0, o1, si0, si1, so0, so1):
    ins, outs = [in0, in1], [o0, o1]
    sis, sos = [si0, si1], [so0, so1]
    wid = lax.axis_index("s") * NC + lax.axis_index("c")
    s0 = SLABS_BASE * wid + jnp.minimum(wid, EXTRA)
    r0 = ROWS_TC + s0 * 8
    q0 = s0 * 8
    has_tail = wid < EXTRA
    lane = lax.iota(jnp.int32, 16)
    quad = lax.shift_right_logical(lane, 2)
    cphase = [lane * 4 + ((quad + g) & 3) for g in range(4)]

    def in_copy(i, b):
        return pltpu.make_async_copy(
            x_hbm.at[pl.ds(r0 + i * R_CHUNK, R_CHUNK)], ins[b], sis[b])

    def out_copy(i, b):
        return pltpu.make_async_copy(
            outs[b], out_hbm.at[pl.ds(q0 + i * R_CHUNK, R_CHUNK)], sos[b])

    def compute(b, nv):
        in_v, out_v = ins[b], outs[b]

        @plsc.parallel_loop(0, nv, 1, unroll=8)
        def step(v):
            row = v >> 3
            j = v & 7
            ridx = jnp.full((16,), row, jnp.int32)
            coff = j * 64
            a = (plsc.load_gather(in_v, [ridx, cphase[0] + coff])
                 + plsc.load_gather(in_v, [ridx, cphase[1] + coff])
                 + plsc.load_gather(in_v, [ridx, cphase[2] + coff])
                 + plsc.load_gather(in_v, [ridx, cphase[3] + coff]))
            out_v[row, pl.ds(j * 16, 16)] = a

    in_copy(0, 0).start()
    in_copy(1, 1).start()

    def pair(g2, carry):
        for b in range(2):
            i = 2 * g2 + b

            @pl.when(i >= 2)
            def _wait_outbuf():
                out_copy(i - 2, b).wait()

            in_copy(i, b).wait()
            compute(b, VPER)
            out_copy(i, b).start()

            @pl.when(i + 2 < NCHUNK)
            def _prefetch():
                in_copy(i + 2, b).start()

        return carry

    lax.fori_loop(0, NCHUNK // 2, pair, 0)

    out_copy(NCHUNK - 2, 0).wait()

    @pl.when(has_tail)
    def _tail():
        tr0 = r0 + NCHUNK * R_CHUNK
        tq0 = q0 + NCHUNK * R_CHUNK
        tin = pltpu.make_async_copy(
            x_hbm.at[pl.ds(tr0, 8)], ins[0].at[pl.ds(0, 8)], sis[0])
        tin.start()
        tin.wait()
        compute(0, VPER_TAIL)
        tout = pltpu.make_async_copy(
            outs[0].at[pl.ds(0, 8)], out_hbm.at[pl.ds(tq0, 8)], sos[0])
        tout.start()
        tout.wait()

    out_copy(NCHUNK - 1, 1).wait()


_SEL = np.zeros((SIZE_IN_K, SIZE_OUT_K), np.float32)
_SEL[np.arange(SIZE_IN_K), np.arange(SIZE_IN_K) // 4] = 1.0


def _tc_body(x_ref, s_ref, o_ref):
    o_ref[...] = jnp.dot(x_ref[...], s_ref[...],
                         preferred_element_type=jnp.float32)


_tc_seg_sum = pl.pallas_call(
    _tc_body,
    grid=(ROWS_TC // BLOCK_R,),
    in_specs=[
        pl.BlockSpec((BLOCK_R, SIZE_IN_K), lambda i: (i, 0)),
        pl.BlockSpec((SIZE_IN_K, SIZE_OUT_K), lambda i: (0, 0)),
    ],
    out_specs=pl.BlockSpec((BLOCK_R, SIZE_OUT_K), lambda i: (i, 0)),
    out_shape=jax.ShapeDtypeStruct((ROWS_TC, SIZE_OUT_K), jnp.float32),
)


def kernel(x):
    out_sc = _sc_seg_sum(x)
    out_tc = _tc_seg_sum(x, jnp.asarray(_SEL))
    return jnp.concatenate([out_tc, out_sc], axis=0)
